# baseline (speedup 1.0000x reference)
import functools

import jax
import jax.numpy as jnp
from jax import lax
from jax.experimental import pallas as pl
from jax.experimental.pallas import tpu as pltpu
from jax.experimental.pallas import tpu_sc as plsc

B = 16384
S = 20
D = 64
L = 16
NC, NS = 2, 16
NW = NC * NS

CB = 32
K = CB * S
G = 128
NG = K // G
BATCH_PER_W = B // NW
CHUNKS = BATCH_PER_W // CB
ROWS_PER_W = BATCH_PER_W * S

_mesh = plsc.VectorSubcoreMesh(
    core_axis_name="c", subcore_axis_name="s", num_cores=NC, num_subcores=NS
)


@functools.partial(
    pl.kernel,
    out_type=jax.ShapeDtypeStruct((B, D), jnp.float32),
    mesh=_mesh,
    scratch_types=[
        pltpu.VMEM((K,), jnp.int32),
        pltpu.VMEM((K, D), jnp.float32),
        pltpu.VMEM((CB, D), jnp.float32),
        pltpu.SemaphoreType.DMA,
    ],
    compiler_params=pltpu.CompilerParams(use_tc_tiling_on_sc=False),
)
def _embed_mean(labels_hbm, table_hbm, out_hbm, idx_v, rows_v, out_v, sem):
    wid = lax.axis_index("s") * NC + lax.axis_index("c")

    def chunk_body(ci, carry):
        rowbase = wid * ROWS_PER_W + ci * K
        pltpu.sync_copy(labels_hbm.at[pl.ds(rowbase, K)], idx_v)
        copies = []
        for j in range(NG):
            copies.append(
                pltpu.async_copy(
                    table_hbm.at[idx_v.at[pl.ds(j * G, G)]],
                    rows_v.at[pl.ds(j * G, G)],
                    sem,
                )
            )
        for c in copies:
            c.wait()

        def batch_body(b, carry2):
            rbase = b * S
            for d in range(D // L):
                sl = pl.ds(d * L, L)
                acc = rows_v[rbase, sl]
                for s2 in range(1, S):
                    acc = acc + rows_v[rbase + s2, sl]
                out_v[b, sl] = acc * (1.0 / S)
            return carry2

        lax.fori_loop(0, CB, batch_body, 0)
        pltpu.sync_copy(out_v, out_hbm.at[pl.ds(wid * BATCH_PER_W + ci * CB, CB)])
        return carry

    lax.fori_loop(0, CHUNKS, chunk_body, 0)


def kernel(labels, table):
    idx = labels.astype(jnp.int32).reshape(B * S)
    return _embed_mean(idx, table)
